# baseline (speedup 1.0000x reference)
import jax
import jax.numpy as jnp
from jax.experimental import pallas as pl
from jax.experimental.pallas import tpu as pltpu

B, C, H, W_ = 8, 96, 32, 32
N = H * W_
K = 9
G = 4
CG = C // G
EPS_GN = 1e-5
NEG = -3.0e38
QUARTER = N // 4


def _gcn_block_kernel(x_ref, w1_ref, b1_ref, w2_ref, b2_ref,
                      g1w_ref, g1b_ref, g2w_ref, g2b_ref, out_ref):
    xf = x_ref[0]

    nrm = jnp.sqrt(jnp.sum(xf * xf, axis=1, keepdims=True))
    xn = xf / jnp.maximum(nrm, 1e-12)

    xt1 = jnp.dot(xn, w1_ref[...], preferred_element_type=jnp.float32)

    def topk_weights(rows):
        sim = jax.lax.dot_general(
            rows, xn, dimension_numbers=(((1,), (1,)), ((), ())),
            preferred_element_type=jnp.float32)
        p = jnp.maximum(sim[:, :N // 2], sim[:, N // 2:])
        q = jnp.minimum(sim[:, :N // 2], sim[:, N // 2:])
        m = jnp.max(p, axis=1, keepdims=True)
        deg = m
        for _ in range(K - 1):
            ext = p >= m
            p = jnp.where(ext, q, p)
            q = jnp.where(ext, NEG, q)
            m = jnp.max(p, axis=1, keepdims=True)
            deg = deg + m
        w = jnp.where(sim >= m, sim, 0.0)
        return w, deg + 1e-6

    parts = [topk_weights(xn[i * QUARTER:(i + 1) * QUARTER])
             for i in range(4)]

    gid = jax.lax.broadcasted_iota(jnp.int32, (1, C), 1) // CG

    def aggregate(xt, b_ref):
        hs = [jnp.dot(w, xt, preferred_element_type=jnp.float32) / deg
              for w, deg in parts]
        return jnp.concatenate(hs, axis=0) + b_ref[...]

    def gn_silu(h, gw_ref, gb_ref):
        s = jnp.sum(h, axis=0, keepdims=True)
        ss = jnp.sum(h * h, axis=0, keepdims=True)
        cnt = float(N * CG)
        mean_c = jnp.zeros((1, C), jnp.float32)
        ex2_c = jnp.zeros((1, C), jnp.float32)
        for g in range(G):
            mask = gid == g
            mg = jnp.sum(jnp.where(mask, s, 0.0)) / cnt
            eg = jnp.sum(jnp.where(mask, ss, 0.0)) / cnt
            mean_c = jnp.where(mask, mg, mean_c)
            ex2_c = jnp.where(mask, eg, ex2_c)
        var_c = ex2_c - mean_c * mean_c
        inv = jax.lax.rsqrt(var_c + EPS_GN)
        hn = (h - mean_c) * inv * gw_ref[...] + gb_ref[...]
        return hn * jax.nn.sigmoid(hn)

    s1 = gn_silu(aggregate(xt1, b1_ref), g1w_ref, g1b_ref)
    xt2 = jnp.dot(s1, w2_ref[...], preferred_element_type=jnp.float32)
    out_ref[0] = gn_silu(aggregate(xt2, b2_ref), g2w_ref, g2b_ref)


def kernel(x, W1, b1, W2, b2, gn1_w, gn1_b, gn2_w, gn2_b):
    xf = x.reshape(B, C, N).transpose(0, 2, 1)
    vec = lambda v: v.reshape(1, C)
    full = lambda shape: pl.BlockSpec(shape, lambda b: (0,) * len(shape))

    y = pl.pallas_call(
        _gcn_block_kernel,
        grid=(B,),
        in_specs=[
            pl.BlockSpec((1, N, C), lambda b: (b, 0, 0)),
            full((C, C)), full((1, C)),
            full((C, C)), full((1, C)),
            full((1, C)), full((1, C)),
            full((1, C)), full((1, C)),
        ],
        out_specs=pl.BlockSpec((1, N, C), lambda b: (b, 0, 0)),
        out_shape=jax.ShapeDtypeStruct((B, N, C), jnp.float32),
        compiler_params=pltpu.CompilerParams(
            dimension_semantics=("arbitrary",)),
    )(xf, W1, vec(b1), W2, vec(b2),
      vec(gn1_w), vec(gn1_b), vec(gn2_w), vec(gn2_b))

    return y.transpose(0, 2, 1).reshape(B, C, H, W_)


# speedup vs baseline: 1.0696x; 1.0696x over previous
import jax
import jax.numpy as jnp
from jax.experimental import pallas as pl
from jax.experimental.pallas import tpu as pltpu

B, C, H, W_ = 8, 96, 32, 32
N = H * W_
K = 9
G = 4
CG = C // G
EPS_GN = 1e-5
NEG = -3.0e38
QUARTER = N // 4
PER_STEP = 2


def _one_batch(xf, w1_ref, b1_ref, w2_ref, b2_ref,
               g1w_ref, g1b_ref, g2w_ref, g2b_ref):
    nrm = jnp.sqrt(jnp.sum(xf * xf, axis=1, keepdims=True))
    xn = xf / jnp.maximum(nrm, 1e-12)

    xt1 = jnp.dot(xn, w1_ref[...], preferred_element_type=jnp.float32)

    def topk_weights(rows):
        sim = jax.lax.dot_general(
            rows, xn, dimension_numbers=(((1,), (1,)), ((), ())),
            preferred_element_type=jnp.float32)
        m = jnp.max(sim, axis=1, keepdims=True)
        deg = m
        for _ in range(K - 1):
            m = jnp.max(jnp.where(sim < m, sim, NEG), axis=1, keepdims=True)
            deg = deg + m
        w = jnp.where(sim >= m, sim, 0.0)
        return w, deg + 1e-6

    parts = [topk_weights(xn[i * QUARTER:(i + 1) * QUARTER])
             for i in range(4)]

    gid = jax.lax.broadcasted_iota(jnp.int32, (1, C), 1) // CG

    def aggregate(xt, b_ref):
        hs = [jnp.dot(w, xt, preferred_element_type=jnp.float32) / deg
              for w, deg in parts]
        return jnp.concatenate(hs, axis=0) + b_ref[...]

    def gn_silu(h, gw_ref, gb_ref):
        s = jnp.sum(h, axis=0, keepdims=True)
        ss = jnp.sum(h * h, axis=0, keepdims=True)
        cnt = float(N * CG)
        mean_c = jnp.zeros((1, C), jnp.float32)
        ex2_c = jnp.zeros((1, C), jnp.float32)
        for g in range(G):
            mask = gid == g
            mg = jnp.sum(jnp.where(mask, s, 0.0)) / cnt
            eg = jnp.sum(jnp.where(mask, ss, 0.0)) / cnt
            mean_c = jnp.where(mask, mg, mean_c)
            ex2_c = jnp.where(mask, eg, ex2_c)
        var_c = ex2_c - mean_c * mean_c
        inv = jax.lax.rsqrt(var_c + EPS_GN)
        hn = (h - mean_c) * inv * gw_ref[...] + gb_ref[...]
        return hn * jax.nn.sigmoid(hn)

    s1 = gn_silu(aggregate(xt1, b1_ref), g1w_ref, g1b_ref)
    xt2 = jnp.dot(s1, w2_ref[...], preferred_element_type=jnp.float32)
    return gn_silu(aggregate(xt2, b2_ref), g2w_ref, g2b_ref)


def _gcn_block_kernel(x_ref, w1_ref, b1_ref, w2_ref, b2_ref,
                      g1w_ref, g1b_ref, g2w_ref, g2b_ref, out_ref):
    for bi in range(PER_STEP):
        out_ref[bi] = _one_batch(
            x_ref[bi], w1_ref, b1_ref, w2_ref, b2_ref,
            g1w_ref, g1b_ref, g2w_ref, g2b_ref)


def kernel(x, W1, b1, W2, b2, gn1_w, gn1_b, gn2_w, gn2_b):
    xf = x.reshape(B, C, N).transpose(0, 2, 1)
    vec = lambda v: v.reshape(1, C)
    full = lambda shape: pl.BlockSpec(shape, lambda b: (0,) * len(shape))

    y = pl.pallas_call(
        _gcn_block_kernel,
        grid=(B // PER_STEP,),
        in_specs=[
            pl.BlockSpec((PER_STEP, N, C), lambda b: (b, 0, 0)),
            full((C, C)), full((1, C)),
            full((C, C)), full((1, C)),
            full((1, C)), full((1, C)),
            full((1, C)), full((1, C)),
        ],
        out_specs=pl.BlockSpec((PER_STEP, N, C), lambda b: (b, 0, 0)),
        out_shape=jax.ShapeDtypeStruct((B, N, C), jnp.float32),
        compiler_params=pltpu.CompilerParams(
            dimension_semantics=("arbitrary",)),
    )(xf, W1, vec(b1), W2, vec(b2),
      vec(gn1_w), vec(gn1_b), vec(gn2_w), vec(gn2_b))

    return y.transpose(0, 2, 1).reshape(B, C, H, W_)


# speedup vs baseline: 1.0717x; 1.0020x over previous
import jax
import jax.numpy as jnp
from jax.experimental import pallas as pl
from jax.experimental.pallas import tpu as pltpu

B, C, H, W_ = 8, 96, 32, 32
N = H * W_
K = 9
G = 4
CG = C // G
EPS_GN = 1e-5
NEG = -3.0e38
QUARTER = N // 4
PER_STEP = 4


def _one_batch(xf, w1_ref, b1_ref, w2_ref, b2_ref,
               g1w_ref, g1b_ref, g2w_ref, g2b_ref):
    nrm = jnp.sqrt(jnp.sum(xf * xf, axis=1, keepdims=True))
    xn = xf / jnp.maximum(nrm, 1e-12)

    xt1 = jnp.dot(xn, w1_ref[...], preferred_element_type=jnp.float32)

    def topk_weights(rows):
        sim = jax.lax.dot_general(
            rows, xn, dimension_numbers=(((1,), (1,)), ((), ())),
            preferred_element_type=jnp.float32)
        m = jnp.max(sim, axis=1, keepdims=True)
        deg = m
        for _ in range(K - 1):
            m = jnp.max(jnp.where(sim < m, sim, NEG), axis=1, keepdims=True)
            deg = deg + m
        w = jnp.where(sim >= m, sim, 0.0)
        return w, deg + 1e-6

    parts = [topk_weights(xn[i * QUARTER:(i + 1) * QUARTER])
             for i in range(4)]

    gid = jax.lax.broadcasted_iota(jnp.int32, (1, C), 1) // CG

    def aggregate(xt, b_ref):
        hs = [jnp.dot(w, xt, preferred_element_type=jnp.float32) / deg
              for w, deg in parts]
        return jnp.concatenate(hs, axis=0) + b_ref[...]

    def gn_silu(h, gw_ref, gb_ref):
        s = jnp.sum(h, axis=0, keepdims=True)
        ss = jnp.sum(h * h, axis=0, keepdims=True)
        cnt = float(N * CG)
        mean_c = jnp.zeros((1, C), jnp.float32)
        ex2_c = jnp.zeros((1, C), jnp.float32)
        for g in range(G):
            mask = gid == g
            mg = jnp.sum(jnp.where(mask, s, 0.0)) / cnt
            eg = jnp.sum(jnp.where(mask, ss, 0.0)) / cnt
            mean_c = jnp.where(mask, mg, mean_c)
            ex2_c = jnp.where(mask, eg, ex2_c)
        var_c = ex2_c - mean_c * mean_c
        inv = jax.lax.rsqrt(var_c + EPS_GN)
        hn = (h - mean_c) * inv * gw_ref[...] + gb_ref[...]
        return hn * jax.nn.sigmoid(hn)

    s1 = gn_silu(aggregate(xt1, b1_ref), g1w_ref, g1b_ref)
    xt2 = jnp.dot(s1, w2_ref[...], preferred_element_type=jnp.float32)
    return gn_silu(aggregate(xt2, b2_ref), g2w_ref, g2b_ref)


def _gcn_block_kernel(x_ref, w1_ref, b1_ref, w2_ref, b2_ref,
                      g1w_ref, g1b_ref, g2w_ref, g2b_ref, out_ref):
    for bi in range(PER_STEP):
        out_ref[bi] = _one_batch(
            x_ref[bi], w1_ref, b1_ref, w2_ref, b2_ref,
            g1w_ref, g1b_ref, g2w_ref, g2b_ref)


def kernel(x, W1, b1, W2, b2, gn1_w, gn1_b, gn2_w, gn2_b):
    xf = x.reshape(B, C, N).transpose(0, 2, 1)
    vec = lambda v: v.reshape(1, C)
    full = lambda shape: pl.BlockSpec(shape, lambda b: (0,) * len(shape))

    y = pl.pallas_call(
        _gcn_block_kernel,
        grid=(B // PER_STEP,),
        in_specs=[
            pl.BlockSpec((PER_STEP, N, C), lambda b: (b, 0, 0)),
            full((C, C)), full((1, C)),
            full((C, C)), full((1, C)),
            full((1, C)), full((1, C)),
            full((1, C)), full((1, C)),
        ],
        out_specs=pl.BlockSpec((PER_STEP, N, C), lambda b: (b, 0, 0)),
        out_shape=jax.ShapeDtypeStruct((B, N, C), jnp.float32),
        compiler_params=pltpu.CompilerParams(
            dimension_semantics=("arbitrary",)),
    )(xf, W1, vec(b1), W2, vec(b2),
      vec(gn1_w), vec(gn1_b), vec(gn2_w), vec(gn2_b))

    return y.transpose(0, 2, 1).reshape(B, C, H, W_)
